# baseline (speedup 1.0000x reference)
import functools

import jax
import jax.numpy as jnp
from jax import lax
from jax.experimental import pallas as pl
from jax.experimental.pallas import tpu as pltpu
from jax.experimental.pallas import tpu_sc as plsc

N = 16777216
BINS = 256
NW = 32
PER_W = N // NW
CHUNK = 16384
N_CHUNKS = PER_W // CHUNK
LANES = 16
UNROLL = 4


def _make_kernel():
    mesh = plsc.VectorSubcoreMesh(core_axis_name="c", subcore_axis_name="s")

    @functools.partial(
        pl.kernel,
        mesh=mesh,
        out_type=jax.ShapeDtypeStruct((N,), jnp.int32),
        compiler_params=pltpu.CompilerParams(needs_layout_passes=False),
        scratch_types=[
            pltpu.VMEM((BINS,), jnp.float32),
            pltpu.VMEM((CHUNK,), jnp.float32),
            pltpu.VMEM((CHUNK,), jnp.float32),
            pltpu.VMEM((CHUNK,), jnp.int32),
            pltpu.VMEM((CHUNK,), jnp.int32),
            pltpu.SemaphoreType.DMA,
            pltpu.SemaphoreType.DMA,
            pltpu.SemaphoreType.DMA,
            pltpu.SemaphoreType.DMA,
        ],
    )
    def bucketize(x_hbm, b_hbm, out_hbm, bbuf, xbuf0, xbuf1,
                  obuf0, obuf1, isem0, isem1, osem0, osem1):
        xbuf = (xbuf0, xbuf1)
        obuf = (obuf0, obuf1)
        isem = (isem0, isem1)
        osem = (osem0, osem1)
        wid = lax.axis_index("s") * 2 + lax.axis_index("c")
        base = wid * PER_W

        def start_in(c, b):
            pltpu.async_copy(
                x_hbm.at[pl.ds(base + c * CHUNK, CHUNK)], xbuf[b], isem[b])

        def wait_in(b):
            pltpu.make_async_copy(
                x_hbm.at[pl.ds(base, CHUNK)], xbuf[b], isem[b]).wait()

        def start_out(c, b):
            pltpu.async_copy(
                obuf[b], out_hbm.at[pl.ds(base + c * CHUNK, CHUNK)], osem[b])

        def wait_out(b):
            pltpu.make_async_copy(
                obuf[b], out_hbm.at[pl.ds(base, CHUNK)], osem[b]).wait()

        start_in(0, 0)
        start_in(1, 1)
        pltpu.sync_copy(b_hbm, bbuf)

        b_lo = bbuf[pl.ds(0, LANES)][0]
        b_hi = bbuf[pl.ds(BINS - LANES, LANES)][LANES - 1]
        d = b_hi - b_lo
        r = lax.bitcast_convert_type(
            jnp.int32(0x7EF311C3) - lax.bitcast_convert_type(d, jnp.int32),
            jnp.float32)
        r = r * (2.0 - d * r)
        r = r * (2.0 - d * r)
        r = r * (2.0 - d * r)
        inv = (BINS - 1.0) * r
        off = 0.5 - b_lo * inv

        def outer(g, carry):
            for b in range(2):
                c = g * 2 + b
                wait_in(b)

                @pl.when(c >= 2)
                def _():
                    wait_out(b)

                @plsc.parallel_loop(0, CHUNK // LANES, unroll=UNROLL)
                def _(i):
                    xv = xbuf[b][pl.ds(i * LANES, LANES)]
                    u = xv * inv + off
                    u = jnp.minimum(jnp.maximum(u, 0.0), BINS - 1.0)
                    g16 = u.astype(jnp.int32)
                    bg = plsc.load_gather(bbuf, [g16])
                    obuf[b][pl.ds(i * LANES, LANES)] = (
                        g16 + (bg < xv).astype(jnp.int32))

                start_out(c, b)

                @pl.when(c + 2 < N_CHUNKS)
                def _():
                    start_in(c + 2, b)
            return carry

        lax.fori_loop(0, N_CHUNKS // 2, outer, 0)
        wait_out(0)
        wait_out(1)

    return bucketize


_BUCKETIZE = _make_kernel()


def kernel(x, boundaries):
    return _BUCKETIZE(x, boundaries)


# speedup vs baseline: 1.0519x; 1.0519x over previous
import functools

import jax
import jax.numpy as jnp
from jax import lax
from jax.experimental import pallas as pl
from jax.experimental.pallas import tpu as pltpu
from jax.experimental.pallas import tpu_sc as plsc

N = 16777216
BINS = 256
NW = 32
PER_W = N // NW
CHUNK = 16384
N_CHUNKS = PER_W // CHUNK
LANES = 16
UNROLL = 8


def _make_kernel():
    mesh = plsc.VectorSubcoreMesh(core_axis_name="c", subcore_axis_name="s")

    @functools.partial(
        pl.kernel,
        mesh=mesh,
        out_type=jax.ShapeDtypeStruct((N,), jnp.int32),
        compiler_params=pltpu.CompilerParams(needs_layout_passes=False),
        scratch_types=[
            pltpu.VMEM((BINS,), jnp.float32),
            pltpu.VMEM((CHUNK,), jnp.float32),
            pltpu.VMEM((CHUNK,), jnp.float32),
            pltpu.VMEM((CHUNK,), jnp.int32),
            pltpu.VMEM((CHUNK,), jnp.int32),
            pltpu.SemaphoreType.DMA,
            pltpu.SemaphoreType.DMA,
            pltpu.SemaphoreType.DMA,
            pltpu.SemaphoreType.DMA,
        ],
    )
    def bucketize(x_hbm, b_hbm, out_hbm, bbuf, xbuf0, xbuf1,
                  obuf0, obuf1, isem0, isem1, osem0, osem1):
        xbuf = (xbuf0, xbuf1)
        obuf = (obuf0, obuf1)
        isem = (isem0, isem1)
        osem = (osem0, osem1)
        wid = lax.axis_index("s") * 2 + lax.axis_index("c")
        base = wid * PER_W

        def start_in(c, b):
            pltpu.async_copy(
                x_hbm.at[pl.ds(base + c * CHUNK, CHUNK)], xbuf[b], isem[b])

        def wait_in(b):
            pltpu.make_async_copy(
                x_hbm.at[pl.ds(base, CHUNK)], xbuf[b], isem[b]).wait()

        def start_out(c, b):
            pltpu.async_copy(
                obuf[b], out_hbm.at[pl.ds(base + c * CHUNK, CHUNK)], osem[b])

        def wait_out(b):
            pltpu.make_async_copy(
                obuf[b], out_hbm.at[pl.ds(base, CHUNK)], osem[b]).wait()

        start_in(0, 0)
        start_in(1, 1)
        pltpu.sync_copy(b_hbm, bbuf)

        b_lo = bbuf[pl.ds(0, LANES)][0]
        b_hi = bbuf[pl.ds(BINS - LANES, LANES)][LANES - 1]
        d = b_hi - b_lo
        r = lax.bitcast_convert_type(
            jnp.int32(0x7EF311C3) - lax.bitcast_convert_type(d, jnp.int32),
            jnp.float32)
        r = r * (2.0 - d * r)
        r = r * (2.0 - d * r)
        r = r * (2.0 - d * r)
        inv = (BINS - 1.0) * r
        off = 0.5 - b_lo * inv

        def outer(g, carry):
            for b in range(2):
                c = g * 2 + b
                wait_in(b)

                @pl.when(c >= 2)
                def _():
                    wait_out(b)

                @plsc.parallel_loop(0, CHUNK // LANES, unroll=UNROLL)
                def _(i):
                    xv = xbuf[b][pl.ds(i * LANES, LANES)]
                    u = xv * inv + off
                    u = jnp.minimum(jnp.maximum(u, 0.0), BINS - 1.0)
                    g16 = u.astype(jnp.int32)
                    bg = plsc.load_gather(bbuf, [g16])
                    obuf[b][pl.ds(i * LANES, LANES)] = g16
                    plsc.addupdate(obuf[b].at[pl.ds(i * LANES, LANES)],
                                   (bg < xv).astype(jnp.int32))

                start_out(c, b)

                @pl.when(c + 2 < N_CHUNKS)
                def _():
                    start_in(c + 2, b)
            return carry

        lax.fori_loop(0, N_CHUNKS // 2, outer, 0)
        wait_out(0)
        wait_out(1)

    return bucketize


_BUCKETIZE = _make_kernel()


def kernel(x, boundaries):
    return _BUCKETIZE(x, boundaries)


# speedup vs baseline: 1.1035x; 1.0490x over previous
import functools

import jax
import jax.numpy as jnp
from jax import lax
from jax.experimental import pallas as pl
from jax.experimental.pallas import tpu as pltpu
from jax.experimental.pallas import tpu_sc as plsc

N = 16777216
BINS = 256
NW = 32
PER_W = N // NW
CHUNK = 16384
N_CHUNKS = PER_W // CHUNK
LANES = 16
UNROLL = 8


def _make_kernel():
    mesh = plsc.VectorSubcoreMesh(core_axis_name="c", subcore_axis_name="s")

    @functools.partial(
        pl.kernel,
        mesh=mesh,
        out_type=jax.ShapeDtypeStruct((N,), jnp.int32),
        compiler_params=pltpu.CompilerParams(needs_layout_passes=False),
        scratch_types=[
            pltpu.VMEM((BINS,), jnp.float32),
            pltpu.VMEM((CHUNK,), jnp.float32),
            pltpu.VMEM((CHUNK,), jnp.float32),
            pltpu.VMEM((CHUNK,), jnp.int32),
            pltpu.VMEM((CHUNK,), jnp.int32),
            pltpu.SemaphoreType.DMA,
            pltpu.SemaphoreType.DMA,
            pltpu.SemaphoreType.DMA,
            pltpu.SemaphoreType.DMA,
        ],
    )
    def bucketize(x_hbm, b_hbm, out_hbm, bbuf, xbuf0, xbuf1,
                  obuf0, obuf1, isem0, isem1, osem0, osem1):
        xbuf = (xbuf0, xbuf1)
        obuf = (obuf0, obuf1)
        isem = (isem0, isem1)
        osem = (osem0, osem1)
        wid = lax.axis_index("s") * 2 + lax.axis_index("c")
        base = wid * PER_W

        def start_in(c, b):
            pltpu.async_copy(
                x_hbm.at[pl.ds(base + c * CHUNK, CHUNK)], xbuf[b], isem[b])

        def wait_in(b):
            pltpu.make_async_copy(
                x_hbm.at[pl.ds(base, CHUNK)], xbuf[b], isem[b]).wait()

        def start_out(c, b):
            pltpu.async_copy(
                obuf[b], out_hbm.at[pl.ds(base + c * CHUNK, CHUNK)], osem[b])

        def wait_out(b):
            pltpu.make_async_copy(
                obuf[b], out_hbm.at[pl.ds(base, CHUNK)], osem[b]).wait()

        start_in(0, 0)
        start_in(1, 1)
        pltpu.sync_copy(b_hbm, bbuf)

        b_lo = bbuf[pl.ds(0, LANES)][0]
        b_hi = bbuf[pl.ds(BINS - LANES, LANES)][LANES - 1]
        d = b_hi - b_lo
        r = lax.bitcast_convert_type(
            jnp.int32(0x7EF311C3) - lax.bitcast_convert_type(d, jnp.int32),
            jnp.float32)
        r = r * (2.0 - d * r)
        r = r * (2.0 - d * r)
        r = r * (2.0 - d * r)
        inv = (BINS - 1.0) * r
        off = 0.5 - b_lo * inv

        def outer(g, carry):
            for b in range(2):
                c = g * 2 + b
                wait_in(b)

                @pl.when(c >= 2)
                def _():
                    wait_out(b)

                @plsc.parallel_loop(0, CHUNK // LANES, unroll=UNROLL)
                def _(i):
                    xv = xbuf[b][pl.ds(i * LANES, LANES)]
                    u = xv * inv + off
                    u = jnp.minimum(jnp.maximum(u, 0.0), BINS - 1.0)
                    g16 = u.astype(jnp.int32)
                    bg = plsc.load_gather(bbuf, [g16])
                    obuf[b][pl.ds(i * LANES, LANES)] = (
                        g16 + (bg < xv).astype(jnp.int32))

                start_out(c, b)

                @pl.when(c + 2 < N_CHUNKS)
                def _():
                    start_in(c + 2, b)
            return carry

        lax.fori_loop(0, N_CHUNKS // 2, outer, 0)
        wait_out(0)
        wait_out(1)

    return bucketize


_BUCKETIZE = _make_kernel()


def kernel(x, boundaries):
    return _BUCKETIZE(x, boundaries)
